# baseline (speedup 1.0000x reference)
import functools

import jax
import jax.numpy as jnp
from jax import lax
from jax.experimental import pallas as pl
from jax.experimental.pallas import tpu as pltpu
from jax.experimental.pallas import tpu_sc as plsc

N = 10000
E = 320000
D_IN = 128
HID = 128
NUM_CLASSES = 40

NC = 2
NS = 16
NW = NC * NS
CH = 128
NR = 5
NRH = 2
NCHUNK = 80
EPT = CH * NCHUNK
E_PAD = EPT * NW
HW = 64
N_PAD = 10240
C_PAD = 48
ROWS_PT = N_PAD // NS

_mesh = plsc.VectorSubcoreMesh(core_axis_name="c", subcore_axis_name="s")
_sc_params = pltpu.CompilerParams(needs_layout_passes=False, use_tc_tiling_on_sc=False)


@functools.partial(
    pl.kernel,
    out_type=jax.ShapeDtypeStruct((NW, N_PAD), jnp.float32),
    mesh=_mesh,
    compiler_params=_sc_params,
    scratch_types=[
        pltpu.VMEM((N_PAD,), jnp.float32),
        pltpu.VMEM((EPT,), jnp.int32),
    ],
)
def _deg_sc(dst_hbm, out_hbm, hist_v, dstbuf_v):
    c = lax.axis_index("c")
    s = lax.axis_index("s")
    wid = s * NC + c

    def zero(i, _):
        hist_v[pl.ds(i * 16, 16)] = jnp.zeros((16,), jnp.float32)
        return ()

    lax.fori_loop(0, N_PAD // 16, zero, ())
    pltpu.sync_copy(dst_hbm.at[pl.ds(wid * EPT, EPT)], dstbuf_v)
    ones = jnp.ones((16,), jnp.float32)

    def body(i, _):
        idx = dstbuf_v[pl.ds(i * 16, 16)]
        plsc.addupdate_scatter(hist_v, [idx], ones)
        return ()

    lax.fori_loop(0, EPT // 16, body, ())
    pltpu.sync_copy(hist_v, out_hbm.at[wid])


@functools.partial(
    pl.kernel,
    out_type=tuple(jax.ShapeDtypeStruct((N_PAD, HW), jnp.float32)
                   for _ in range(4)),
    mesh=_mesh,
    compiler_params=_sc_params,
    scratch_types=(
        [pltpu.VMEM((2, CH), jnp.int32) for _ in range(NR)]
        + [pltpu.VMEM((CH, HW), jnp.float32) for _ in range(NR)]
        + [pltpu.VMEM_SHARED((N_PAD, HW), jnp.float32)]
        + [pltpu.VMEM_SHARED((N_PAD, HW), jnp.float32)]
        + [pltpu.SemaphoreType.DMA for _ in range(2 * NR)]
    ),
)
def _agg_sc(ei_hbm, ulo_hbm, uhi_hbm, zeros_hbm,
            o0lo, o0hi, o1lo, o1hi, *scr):
    idx_v = scr[:NR]
    rows_v = scr[NR:2 * NR]
    u_sh = scr[2 * NR]
    acc_sh = scr[2 * NR + 1]
    isem = scr[2 * NR + 2:3 * NR + 2]
    gsem = scr[3 * NR + 2:4 * NR + 2]
    c = lax.axis_index("c")
    s = lax.axis_index("s")
    wid = s * NC + c
    rslice = pl.ds(s * ROWS_PT, ROWS_PT)
    cbase = wid * NCHUNK

    for u_hbm, out0, out1 in ((ulo_hbm, o0lo, o1lo), (uhi_hbm, o0hi, o1hi)):
        usrc = [u_hbm if b < NRH else u_sh for b in range(NR)]
        ucp = pltpu.async_copy(u_hbm.at[rslice], u_sh.at[rslice], isem[0])
        pltpu.sync_copy(zeros_hbm.at[rslice], acc_sh.at[rslice])
        ucp.wait()
        for b in range(NR):
            pltpu.async_copy(ei_hbm.at[cbase + b], idx_v[b], isem[b])
        plsc.subcore_barrier()
        for j in range(NR - 1):
            pltpu.make_async_copy(ei_hbm.at[cbase + j],
                                  idx_v[j], isem[j]).wait()
            pltpu.async_copy(usrc[j].at[idx_v[j].at[0]], rows_v[j], gsem[j])

        def body(q, _):
            for b in range(NR):
                i = q * NR + b
                bg = (b + NR - 1) % NR

                @pl.when(i + NR - 1 < NCHUNK)
                def _():
                    pltpu.make_async_copy(ei_hbm.at[cbase + i + NR - 1],
                                          idx_v[bg], isem[bg]).wait()
                    pltpu.async_copy(usrc[bg].at[idx_v[bg].at[0]],
                                     rows_v[bg], gsem[bg])

                pltpu.make_async_copy(usrc[b].at[idx_v[b].at[0]],
                                      rows_v[b], gsem[b]).wait()
                pltpu.sync_copy(rows_v[b], acc_sh.at[idx_v[b].at[1]], add=True)

                @pl.when(i + NR < NCHUNK)
                def _():
                    pltpu.async_copy(ei_hbm.at[cbase + i + NR],
                                     idx_v[b], isem[b])

            return ()

        lax.fori_loop(0, NCHUNK // NR, body, ())
        plsc.subcore_barrier()

        @pl.when(c == 0)
        def _():
            pltpu.sync_copy(acc_sh.at[rslice], out0.at[rslice])

        @pl.when(c == 1)
        def _():
            pltpu.sync_copy(acc_sh.at[rslice], out1.at[rslice])

        plsc.subcore_barrier()


def _dinv_body(part_ref, out_ref):
    deg = jnp.sum(part_ref[...], axis=0) + 1.0
    out_ref[...] = lax.rsqrt(deg)


def _u1_body(x_ref, w_ref, dinv_ref, lo_ref, hi_ref):
    xw = jnp.dot(x_ref[...], w_ref[...], preferred_element_type=jnp.float32)
    u = xw * dinv_ref[...]
    lo_ref[...] = u[:, :HW]
    hi_ref[...] = u[:, HW:]


def _u2_body(a0lo_ref, a0hi_ref, a1lo_ref, a1hi_ref, u1lo_ref, u1hi_ref,
             dinv_ref, b1lo_ref, b1hi_ref, lo_ref, hi_ref):
    dinv = dinv_ref[...]
    hlo = dinv * (a0lo_ref[...] + a1lo_ref[...] + u1lo_ref[...]) + b1lo_ref[...]
    hhi = dinv * (a0hi_ref[...] + a1hi_ref[...] + u1hi_ref[...]) + b1hi_ref[...]
    lo_ref[...] = dinv * jnp.maximum(hlo, 0.0)
    hi_ref[...] = dinv * jnp.maximum(hhi, 0.0)


def _out_body(c0lo_ref, c0hi_ref, c1lo_ref, c1hi_ref, u2lo_ref, u2hi_ref,
              dinv_ref, w2lo_ref, w2hi_ref, b2_ref, out_ref):
    dinv = dinv_ref[...]
    agg_lo = dinv * (c0lo_ref[...] + c1lo_ref[...] + u2lo_ref[...])
    agg_hi = dinv * (c0hi_ref[...] + c1hi_ref[...] + u2hi_ref[...])
    z = (jnp.dot(agg_lo, w2lo_ref[...], preferred_element_type=jnp.float32)
         + jnp.dot(agg_hi, w2hi_ref[...], preferred_element_type=jnp.float32)
         + b2_ref[...])
    col = lax.broadcasted_iota(jnp.int32, z.shape, 1)
    z = jnp.where(col < NUM_CLASSES, z, -jnp.inf)
    m = jnp.max(z, axis=1, keepdims=True)
    e = jnp.exp(z - m)
    lse = jnp.log(jnp.sum(e, axis=1, keepdims=True))
    out_ref[...] = z - m - lse


_R = 1024
_G = N_PAD // _R


def _row_spec(w):
    return pl.BlockSpec((_R, w), lambda i: (i, 0))


def _const_spec(shape):
    return pl.BlockSpec(shape, lambda i: (0, 0))


def kernel(x, edge_index, W1, b1, W2, b2):
    f32 = jnp.float32
    src = edge_index[0]
    dst = edge_index[1]
    pad_idx = jnp.full((E_PAD - E,), N_PAD - 1, dtype=src.dtype)
    src_p = jnp.concatenate([src, pad_idx])
    dst_p = jnp.concatenate([dst, pad_idx])
    ei3 = jnp.stack([jnp.reshape(src_p, (E_PAD // CH, CH)),
                     jnp.reshape(dst_p, (E_PAD // CH, CH))], axis=1)
    x_p = jnp.pad(x, ((0, N_PAD - N), (0, 0)))
    w2_p = jnp.pad(W2, ((0, 0), (0, C_PAD - NUM_CLASSES)))
    b1lo = jnp.reshape(b1[:HW], (1, HW))
    b1hi = jnp.reshape(b1[HW:], (1, HW))
    b2r = jnp.reshape(jnp.pad(b2, (0, C_PAD - NUM_CLASSES)), (1, C_PAD))
    zeros64 = jnp.zeros((N_PAD, HW), f32)

    deg_part = _deg_sc(dst_p)

    dinv = pl.pallas_call(
        _dinv_body,
        out_shape=jax.ShapeDtypeStruct((N_PAD,), f32),
    )(deg_part)
    dinv2 = jnp.reshape(dinv, (N_PAD, 1))

    u1lo, u1hi = pl.pallas_call(
        _u1_body,
        grid=(_G,),
        in_specs=[_row_spec(D_IN), _const_spec((D_IN, HID)), _row_spec(1)],
        out_specs=(_row_spec(HW), _row_spec(HW)),
        out_shape=(jax.ShapeDtypeStruct((N_PAD, HW), f32),
                   jax.ShapeDtypeStruct((N_PAD, HW), f32)),
    )(x_p, W1, dinv2)

    a0lo, a0hi, a1lo, a1hi = _agg_sc(ei3, u1lo, u1hi, zeros64)

    u2lo, u2hi = pl.pallas_call(
        _u2_body,
        grid=(_G,),
        in_specs=[_row_spec(HW)] * 6
        + [_row_spec(1), _const_spec((1, HW)), _const_spec((1, HW))],
        out_specs=(_row_spec(HW), _row_spec(HW)),
        out_shape=(jax.ShapeDtypeStruct((N_PAD, HW), f32),
                   jax.ShapeDtypeStruct((N_PAD, HW), f32)),
    )(a0lo, a0hi, a1lo, a1hi, u1lo, u1hi, dinv2, b1lo, b1hi)

    c0lo, c0hi, c1lo, c1hi = _agg_sc(ei3, u2lo, u2hi, zeros64)

    out = pl.pallas_call(
        _out_body,
        grid=(_G,),
        in_specs=[_row_spec(HW)] * 6
        + [_row_spec(1), _const_spec((HW, C_PAD)), _const_spec((HW, C_PAD)),
           _const_spec((1, C_PAD))],
        out_specs=_row_spec(C_PAD),
        out_shape=jax.ShapeDtypeStruct((N_PAD, C_PAD), f32),
    )(c0lo, c0hi, c1lo, c1hi, u2lo, u2hi, dinv2,
      w2_p[:HW], w2_p[HW:], b2r)

    return out[:N, :NUM_CLASSES]


# speedup vs baseline: 1.0833x; 1.0833x over previous
import functools

import jax
import jax.numpy as jnp
from jax import lax
from jax.experimental import pallas as pl
from jax.experimental.pallas import tpu as pltpu
from jax.experimental.pallas import tpu_sc as plsc

N = 10000
E = 320000
D_IN = 128
HID = 128
NUM_CLASSES = 40

NC = 2
NS = 16
NW = NC * NS
CH = 128
NR = 5
NRH = 1
NCHUNK = 80
EPT = CH * NCHUNK
E_PAD = EPT * NW
HW = 64
N_PAD = 10240
C_PAD = 48
ROWS_PT = N_PAD // NS

_mesh = plsc.VectorSubcoreMesh(core_axis_name="c", subcore_axis_name="s")
_sc_params = pltpu.CompilerParams(needs_layout_passes=False, use_tc_tiling_on_sc=False)


@functools.partial(
    pl.kernel,
    out_type=jax.ShapeDtypeStruct((NW, N_PAD), jnp.float32),
    mesh=_mesh,
    compiler_params=_sc_params,
    scratch_types=[
        pltpu.VMEM((N_PAD,), jnp.float32),
        pltpu.VMEM((EPT,), jnp.int32),
    ],
)
def _deg_sc(dst_hbm, out_hbm, hist_v, dstbuf_v):
    c = lax.axis_index("c")
    s = lax.axis_index("s")
    wid = s * NC + c

    def zero(i, _):
        hist_v[pl.ds(i * 16, 16)] = jnp.zeros((16,), jnp.float32)
        return ()

    lax.fori_loop(0, N_PAD // 16, zero, ())
    pltpu.sync_copy(dst_hbm.at[pl.ds(wid * EPT, EPT)], dstbuf_v)
    ones = jnp.ones((16,), jnp.float32)

    def body(i, _):
        idx = dstbuf_v[pl.ds(i * 16, 16)]
        plsc.addupdate_scatter(hist_v, [idx], ones)
        return ()

    lax.fori_loop(0, EPT // 16, body, ())
    pltpu.sync_copy(hist_v, out_hbm.at[wid])


@functools.partial(
    pl.kernel,
    out_type=tuple(jax.ShapeDtypeStruct((N_PAD, HW), jnp.float32)
                   for _ in range(4)),
    mesh=_mesh,
    compiler_params=_sc_params,
    scratch_types=(
        [pltpu.VMEM((2, CH), jnp.int32) for _ in range(NR)]
        + [pltpu.VMEM((CH, HW), jnp.float32) for _ in range(NR)]
        + [pltpu.VMEM_SHARED((N_PAD, HW), jnp.float32)]
        + [pltpu.VMEM_SHARED((N_PAD, HW), jnp.float32)]
        + [pltpu.SemaphoreType.DMA for _ in range(2 * NR)]
    ),
)
def _agg_sc(ei_hbm, ulo_hbm, uhi_hbm, zeros_hbm,
            o0lo, o0hi, o1lo, o1hi, *scr):
    idx_v = scr[:NR]
    rows_v = scr[NR:2 * NR]
    u_sh = scr[2 * NR]
    acc_sh = scr[2 * NR + 1]
    isem = scr[2 * NR + 2:3 * NR + 2]
    gsem = scr[3 * NR + 2:4 * NR + 2]
    c = lax.axis_index("c")
    s = lax.axis_index("s")
    wid = s * NC + c
    rslice = pl.ds(s * ROWS_PT, ROWS_PT)
    cbase = wid * NCHUNK

    for u_hbm, out0, out1 in ((ulo_hbm, o0lo, o1lo), (uhi_hbm, o0hi, o1hi)):
        usrc = [u_hbm if b < NRH else u_sh for b in range(NR)]
        ucp = pltpu.async_copy(u_hbm.at[rslice], u_sh.at[rslice], isem[0])
        pltpu.sync_copy(zeros_hbm.at[rslice], acc_sh.at[rslice])
        ucp.wait()
        for b in range(NR):
            pltpu.async_copy(ei_hbm.at[cbase + b], idx_v[b], isem[b])
        plsc.subcore_barrier()
        for j in range(NR - 1):
            pltpu.make_async_copy(ei_hbm.at[cbase + j],
                                  idx_v[j], isem[j]).wait()
            pltpu.async_copy(usrc[j].at[idx_v[j].at[0]], rows_v[j], gsem[j])

        def body(q, _):
            for b in range(NR):
                i = q * NR + b
                bg = (b + NR - 1) % NR

                @pl.when(i + NR - 1 < NCHUNK)
                def _():
                    pltpu.make_async_copy(ei_hbm.at[cbase + i + NR - 1],
                                          idx_v[bg], isem[bg]).wait()
                    pltpu.async_copy(usrc[bg].at[idx_v[bg].at[0]],
                                     rows_v[bg], gsem[bg])

                pltpu.make_async_copy(usrc[b].at[idx_v[b].at[0]],
                                      rows_v[b], gsem[b]).wait()
                pltpu.sync_copy(rows_v[b], acc_sh.at[idx_v[b].at[1]], add=True)

                @pl.when(i + NR < NCHUNK)
                def _():
                    pltpu.async_copy(ei_hbm.at[cbase + i + NR],
                                     idx_v[b], isem[b])

            return ()

        lax.fori_loop(0, NCHUNK // NR, body, ())
        plsc.subcore_barrier()

        @pl.when(c == 0)
        def _():
            pltpu.sync_copy(acc_sh.at[rslice], out0.at[rslice])

        @pl.when(c == 1)
        def _():
            pltpu.sync_copy(acc_sh.at[rslice], out1.at[rslice])

        plsc.subcore_barrier()


def _dinv_body(part_ref, out_ref):
    deg = jnp.sum(part_ref[...], axis=0) + 1.0
    out_ref[...] = lax.rsqrt(deg)


def _u1_body(x_ref, w_ref, dinv_ref, lo_ref, hi_ref):
    xw = jnp.dot(x_ref[...], w_ref[...], preferred_element_type=jnp.float32)
    u = xw * dinv_ref[...]
    lo_ref[...] = u[:, :HW]
    hi_ref[...] = u[:, HW:]


def _u2_body(a0lo_ref, a0hi_ref, a1lo_ref, a1hi_ref, u1lo_ref, u1hi_ref,
             dinv_ref, b1lo_ref, b1hi_ref, lo_ref, hi_ref):
    dinv = dinv_ref[...]
    hlo = dinv * (a0lo_ref[...] + a1lo_ref[...] + u1lo_ref[...]) + b1lo_ref[...]
    hhi = dinv * (a0hi_ref[...] + a1hi_ref[...] + u1hi_ref[...]) + b1hi_ref[...]
    lo_ref[...] = dinv * jnp.maximum(hlo, 0.0)
    hi_ref[...] = dinv * jnp.maximum(hhi, 0.0)


def _out_body(c0lo_ref, c0hi_ref, c1lo_ref, c1hi_ref, u2lo_ref, u2hi_ref,
              dinv_ref, w2lo_ref, w2hi_ref, b2_ref, out_ref):
    dinv = dinv_ref[...]
    agg_lo = dinv * (c0lo_ref[...] + c1lo_ref[...] + u2lo_ref[...])
    agg_hi = dinv * (c0hi_ref[...] + c1hi_ref[...] + u2hi_ref[...])
    z = (jnp.dot(agg_lo, w2lo_ref[...], preferred_element_type=jnp.float32)
         + jnp.dot(agg_hi, w2hi_ref[...], preferred_element_type=jnp.float32)
         + b2_ref[...])
    col = lax.broadcasted_iota(jnp.int32, z.shape, 1)
    z = jnp.where(col < NUM_CLASSES, z, -jnp.inf)
    m = jnp.max(z, axis=1, keepdims=True)
    e = jnp.exp(z - m)
    lse = jnp.log(jnp.sum(e, axis=1, keepdims=True))
    out_ref[...] = z - m - lse


_R = 1024
_G = N_PAD // _R


def _row_spec(w):
    return pl.BlockSpec((_R, w), lambda i: (i, 0))


def _const_spec(shape):
    return pl.BlockSpec(shape, lambda i: (0, 0))


def kernel(x, edge_index, W1, b1, W2, b2):
    f32 = jnp.float32
    src = edge_index[0]
    dst = edge_index[1]
    pad_idx = jnp.full((E_PAD - E,), N_PAD - 1, dtype=src.dtype)
    src_p = jnp.concatenate([src, pad_idx])
    dst_p = jnp.concatenate([dst, pad_idx])
    ei3 = jnp.stack([jnp.reshape(src_p, (E_PAD // CH, CH)),
                     jnp.reshape(dst_p, (E_PAD // CH, CH))], axis=1)
    x_p = jnp.pad(x, ((0, N_PAD - N), (0, 0)))
    w2_p = jnp.pad(W2, ((0, 0), (0, C_PAD - NUM_CLASSES)))
    b1lo = jnp.reshape(b1[:HW], (1, HW))
    b1hi = jnp.reshape(b1[HW:], (1, HW))
    b2r = jnp.reshape(jnp.pad(b2, (0, C_PAD - NUM_CLASSES)), (1, C_PAD))
    zeros64 = jnp.zeros((N_PAD, HW), f32)

    deg_part = _deg_sc(dst_p)

    dinv = pl.pallas_call(
        _dinv_body,
        out_shape=jax.ShapeDtypeStruct((N_PAD,), f32),
    )(deg_part)
    dinv2 = jnp.reshape(dinv, (N_PAD, 1))

    u1lo, u1hi = pl.pallas_call(
        _u1_body,
        grid=(_G,),
        in_specs=[_row_spec(D_IN), _const_spec((D_IN, HID)), _row_spec(1)],
        out_specs=(_row_spec(HW), _row_spec(HW)),
        out_shape=(jax.ShapeDtypeStruct((N_PAD, HW), f32),
                   jax.ShapeDtypeStruct((N_PAD, HW), f32)),
    )(x_p, W1, dinv2)

    a0lo, a0hi, a1lo, a1hi = _agg_sc(ei3, u1lo, u1hi, zeros64)

    u2lo, u2hi = pl.pallas_call(
        _u2_body,
        grid=(_G,),
        in_specs=[_row_spec(HW)] * 6
        + [_row_spec(1), _const_spec((1, HW)), _const_spec((1, HW))],
        out_specs=(_row_spec(HW), _row_spec(HW)),
        out_shape=(jax.ShapeDtypeStruct((N_PAD, HW), f32),
                   jax.ShapeDtypeStruct((N_PAD, HW), f32)),
    )(a0lo, a0hi, a1lo, a1hi, u1lo, u1hi, dinv2, b1lo, b1hi)

    c0lo, c0hi, c1lo, c1hi = _agg_sc(ei3, u2lo, u2hi, zeros64)

    out = pl.pallas_call(
        _out_body,
        grid=(_G,),
        in_specs=[_row_spec(HW)] * 6
        + [_row_spec(1), _const_spec((HW, C_PAD)), _const_spec((HW, C_PAD)),
           _const_spec((1, C_PAD))],
        out_specs=_row_spec(C_PAD),
        out_shape=jax.ShapeDtypeStruct((N_PAD, C_PAD), f32),
    )(c0lo, c0hi, c1lo, c1hi, u2lo, u2hi, dinv2,
      w2_p[:HW], w2_p[HW:], b2r)

    return out[:N, :NUM_CLASSES]
